# initial kernel scaffold (unmeasured)
import jax
import jax.numpy as jnp
from jax import lax
from jax.experimental import pallas as pl
from jax.experimental.pallas import tpu as pltpu

N_DEV = 4
SQ = 2048
SKV = 2048
D_MODEL = 1024
H_PER = 8
DH = 128
QBLK = 256
SCALE = 0.08838834764831843


def _attn_body(x_ref, wq_ref, k_ref, v_ref, wo_ref, out_ref):
    qb_idx = pl.program_id(0)
    h = pl.program_id(1)

    q = jnp.dot(x_ref[...], wq_ref[...], preferred_element_type=jnp.float32)
    k = k_ref[:, 0, :]
    v = v_ref[:, 0, :]
    scores = lax.dot_general(
        q, k, (((1,), (1,)), ((), ())), preferred_element_type=jnp.float32
    ) * SCALE

    row = qb_idx * QBLK + lax.broadcasted_iota(jnp.int32, (QBLK, SKV), 0)
    col = lax.broadcasted_iota(jnp.int32, (QBLK, SKV), 1)
    qb = row // 64
    kb = col // 64
    mask = (qb == kb) | (kb == 0) | (((qb + kb) % 3) == 0)
    scores = jnp.where(mask, scores, -1e9)

    m = jnp.max(scores, axis=1, keepdims=True)
    w = jnp.exp(scores - m)
    w = w / jnp.sum(w, axis=1, keepdims=True)
    ctx = jnp.dot(w, v, preferred_element_type=jnp.float32)
    part = jnp.dot(ctx, wo_ref[...], preferred_element_type=jnp.float32)

    @pl.when(h == 0)
    def _():
        out_ref[...] = part

    @pl.when(h > 0)
    def _():
        out_ref[...] = out_ref[...] + part


def _allreduce_body(p_ref, out_ref, comm_ref, send_sems, recv_sems):
    my = lax.axis_index("i")
    left = (my - 1) % N_DEV
    right = (my + 1) % N_DEV

    barrier = pltpu.get_barrier_semaphore()
    for nbr in (left, right):
        pl.semaphore_signal(
            barrier, inc=1, device_id=(nbr,), device_id_type=pl.DeviceIdType.MESH
        )
    pl.semaphore_wait(barrier, 2)

    out_ref[...] = p_ref[...]
    comm_ref[0, :, :] = p_ref[...]

    for hop in range(N_DEV - 1):
        s = hop % 2
        r = (hop + 1) % 2
        rdma = pltpu.make_async_remote_copy(
            src_ref=comm_ref.at[s],
            dst_ref=comm_ref.at[r],
            send_sem=send_sems.at[s],
            recv_sem=recv_sems.at[r],
            device_id=(right,),
            device_id_type=pl.DeviceIdType.MESH,
        )
        rdma.start()
        rdma.wait()
        out_ref[...] = out_ref[...] + comm_ref[r, :, :]


def kernel(x, Wq, K_ext, V_ext, Wo):
    my = lax.axis_index("i")
    x2d = x.reshape(SQ, D_MODEL)
    Wq_loc = lax.dynamic_slice(Wq, (0, my * (H_PER * DH)), (D_MODEL, H_PER * DH))
    Wo_loc = lax.dynamic_slice(Wo, (my * (H_PER * DH), 0), (H_PER * DH, D_MODEL))
    K = K_ext.reshape(SKV, H_PER, DH)
    V = V_ext.reshape(SKV, H_PER, DH)

    partial = pl.pallas_call(
        _attn_body,
        grid=(SQ // QBLK, H_PER),
        in_specs=[
            pl.BlockSpec((QBLK, D_MODEL), lambda qb, h: (qb, 0)),
            pl.BlockSpec((D_MODEL, DH), lambda qb, h: (0, h)),
            pl.BlockSpec((SKV, 1, DH), lambda qb, h: (0, h, 0)),
            pl.BlockSpec((SKV, 1, DH), lambda qb, h: (0, h, 0)),
            pl.BlockSpec((DH, D_MODEL), lambda qb, h: (h, 0)),
        ],
        out_specs=pl.BlockSpec((QBLK, D_MODEL), lambda qb, h: (qb, 0)),
        out_shape=jax.ShapeDtypeStruct((SQ, D_MODEL), jnp.float32),
    )(x2d, Wq_loc, K, V, Wo_loc)

    out = pl.pallas_call(
        _allreduce_body,
        out_shape=jax.ShapeDtypeStruct((SQ, D_MODEL), jnp.float32),
        in_specs=[pl.BlockSpec(memory_space=pltpu.VMEM)],
        out_specs=pl.BlockSpec(memory_space=pltpu.VMEM),
        scratch_shapes=[
            pltpu.VMEM((2, SQ, D_MODEL), jnp.float32),
            pltpu.SemaphoreType.DMA((2,)),
            pltpu.SemaphoreType.DMA((2,)),
        ],
        compiler_params=pltpu.CompilerParams(collective_id=0),
    )(partial)
    return out.reshape(1, SQ, D_MODEL)


# baseline (device time: 529042 ns/iter reference)
import jax
import jax.numpy as jnp
from jax import lax
from jax.experimental import pallas as pl
from jax.experimental.pallas import tpu as pltpu

N_DEV = 4
SQ = 2048
SKV = 2048
D_MODEL = 1024
H_PER = 8
DH = 128
QBLK = 256
SCALE = 0.08838834764831843


def _attn_body(x_ref, wq_ref, k_ref, v_ref, wo_ref, out_ref):
    qb_idx = pl.program_id(0)
    h = pl.program_id(1)

    q = jnp.dot(x_ref[...], wq_ref[...], preferred_element_type=jnp.float32)
    k = k_ref[0]
    v = v_ref[0]
    scores = lax.dot_general(
        q, k, (((1,), (1,)), ((), ())), preferred_element_type=jnp.float32
    ) * SCALE

    row = qb_idx * QBLK + lax.broadcasted_iota(jnp.int32, (QBLK, SKV), 0)
    col = lax.broadcasted_iota(jnp.int32, (QBLK, SKV), 1)
    qb = row // 64
    kb = col // 64
    mask = (qb == kb) | (kb == 0) | (((qb + kb) % 3) == 0)
    scores = jnp.where(mask, scores, -1e9)

    m = jnp.max(scores, axis=1, keepdims=True)
    w = jnp.exp(scores - m)
    w = w / jnp.sum(w, axis=1, keepdims=True)
    ctx = jnp.dot(w, v, preferred_element_type=jnp.float32)
    part = jnp.dot(ctx, wo_ref[...], preferred_element_type=jnp.float32)

    @pl.when(h == 0)
    def _():
        out_ref[...] = part

    @pl.when(h > 0)
    def _():
        out_ref[...] = out_ref[...] + part


def _allreduce_body(p_ref, out_ref, comm_ref, send_sems, recv_sems):
    my = lax.axis_index("i")
    left = (my - 1) % N_DEV
    right = (my + 1) % N_DEV


    out_ref[...] = p_ref[...]
    comm_ref[0, :, :] = p_ref[...]

    for hop in range(N_DEV - 1):
        s = hop % 2
        r = (hop + 1) % 2
        rdma = pltpu.make_async_remote_copy(
            src_ref=comm_ref.at[s],
            dst_ref=comm_ref.at[r],
            send_sem=send_sems.at[s],
            recv_sem=recv_sems.at[r],
            device_id=(right,),
            device_id_type=pl.DeviceIdType.MESH,
        )
        rdma.start()
        rdma.wait()
        out_ref[...] = out_ref[...] + comm_ref[r, :, :]


def kernel(x, Wq, K_ext, V_ext, Wo):
    my = lax.axis_index("i")
    x2d = x.reshape(SQ, D_MODEL)
    Wq_loc = lax.dynamic_slice(Wq, (0, my * (H_PER * DH)), (D_MODEL, H_PER * DH))
    Wo_loc = lax.dynamic_slice(Wo, (my * (H_PER * DH), 0), (H_PER * DH, D_MODEL))
    K = K_ext.reshape(SKV, H_PER, DH).transpose(1, 0, 2)
    V = V_ext.reshape(SKV, H_PER, DH).transpose(1, 0, 2)

    partial = pl.pallas_call(
        _attn_body,
        grid=(SQ // QBLK, H_PER),
        in_specs=[
            pl.BlockSpec((QBLK, D_MODEL), lambda qb, h: (qb, 0)),
            pl.BlockSpec((D_MODEL, DH), lambda qb, h: (0, h)),
            pl.BlockSpec((1, SKV, DH), lambda qb, h: (h, 0, 0)),
            pl.BlockSpec((1, SKV, DH), lambda qb, h: (h, 0, 0)),
            pl.BlockSpec((DH, D_MODEL), lambda qb, h: (h, 0)),
        ],
        out_specs=pl.BlockSpec((QBLK, D_MODEL), lambda qb, h: (qb, 0)),
        out_shape=jax.ShapeDtypeStruct((SQ, D_MODEL), jnp.float32),
    )(x2d, Wq_loc, K, V, Wo_loc)

    if __import__("os").environ.get("SKIP_AR"):
        return partial.reshape(1, SQ, D_MODEL)
    out = pl.pallas_call(
        _allreduce_body,
        out_shape=jax.ShapeDtypeStruct((SQ, D_MODEL), jnp.float32),
        in_specs=[pl.BlockSpec(memory_space=pltpu.VMEM)],
        out_specs=pl.BlockSpec(memory_space=pltpu.VMEM),
        scratch_shapes=[
            pltpu.VMEM((2, SQ, D_MODEL), jnp.float32),
            pltpu.SemaphoreType.DMA((2,)),
            pltpu.SemaphoreType.DMA((2,)),
        ],
    )(partial)
    return out.reshape(1, SQ, D_MODEL)


# device time: 396262 ns/iter; 1.3351x vs baseline; 1.3351x over previous
import jax
import jax.numpy as jnp
from jax import lax
from jax.experimental import pallas as pl
from jax.experimental.pallas import tpu as pltpu

N_DEV = 4
SQ = 2048
SKV = 2048
D_MODEL = 1024
H_PER = 8
DH = 128
QBLK = 256
SCALE = 0.08838834764831843


def _attn_body(x_ref, wq_ref, k_ref, v_ref, wo_ref, out_ref):
    qb_idx = pl.program_id(0)
    h = pl.program_id(1)

    q = jnp.dot(x_ref[...], wq_ref[...], preferred_element_type=jnp.float32)
    k = k_ref[0]
    v = v_ref[0]
    scores = lax.dot_general(
        q, k, (((1,), (1,)), ((), ())), preferred_element_type=jnp.float32
    ) * SCALE

    row = qb_idx * QBLK + lax.broadcasted_iota(jnp.int32, (QBLK, SKV), 0)
    col = lax.broadcasted_iota(jnp.int32, (QBLK, SKV), 1)
    qb = row // 64
    kb = col // 64
    mask = (qb == kb) | (kb == 0) | (((qb + kb) % 3) == 0)
    scores = jnp.where(mask, scores, -1e9)

    m = jnp.max(scores, axis=1, keepdims=True)
    w = jnp.exp(scores - m)
    w = w / jnp.sum(w, axis=1, keepdims=True)
    ctx = jnp.dot(w, v, preferred_element_type=jnp.float32)
    part = jnp.dot(ctx, wo_ref[...], preferred_element_type=jnp.float32)

    @pl.when(h == 0)
    def _():
        out_ref[...] = part

    @pl.when(h > 0)
    def _():
        out_ref[...] = out_ref[...] + part


CHUNK = SQ // N_DEV


def _allreduce_body(p_ref, out_ref, comm_ref, send_sems, recv_sems):
    my = lax.axis_index("i")
    right = (my + 1) % N_DEV

    out_ref[...] = p_ref[...]

    for s in range(N_DEV - 1):
        c_send = (my + 3 - s) % N_DEV
        c_recv = (my + 2 - s) % N_DEV
        slot = s % 2
        rdma = pltpu.make_async_remote_copy(
            src_ref=out_ref.at[pl.ds(c_send * CHUNK, CHUNK), :],
            dst_ref=comm_ref.at[slot],
            send_sem=send_sems.at[slot],
            recv_sem=recv_sems.at[slot],
            device_id=(right,),
            device_id_type=pl.DeviceIdType.MESH,
        )
        rdma.start()
        rdma.wait()
        out_ref[pl.ds(c_recv * CHUNK, CHUNK), :] += comm_ref[slot]

    for t in range(N_DEV - 1):
        c_send = (my - t) % N_DEV
        slot = (N_DEV - 1 + t) % 2
        rdma = pltpu.make_async_remote_copy(
            src_ref=out_ref.at[pl.ds(c_send * CHUNK, CHUNK), :],
            dst_ref=out_ref.at[pl.ds(c_send * CHUNK, CHUNK), :],
            send_sem=send_sems.at[slot],
            recv_sem=recv_sems.at[slot],
            device_id=(right,),
            device_id_type=pl.DeviceIdType.MESH,
        )
        rdma.start()
        rdma.wait()


def kernel(x, Wq, K_ext, V_ext, Wo):
    my = lax.axis_index("i")
    x2d = x.reshape(SQ, D_MODEL)
    Wq_loc = lax.dynamic_slice(Wq, (0, my * (H_PER * DH)), (D_MODEL, H_PER * DH))
    Wo_loc = lax.dynamic_slice(Wo, (my * (H_PER * DH), 0), (H_PER * DH, D_MODEL))
    K = K_ext.reshape(SKV, H_PER, DH).transpose(1, 0, 2)
    V = V_ext.reshape(SKV, H_PER, DH).transpose(1, 0, 2)

    partial = pl.pallas_call(
        _attn_body,
        grid=(SQ // QBLK, H_PER),
        in_specs=[
            pl.BlockSpec((QBLK, D_MODEL), lambda qb, h: (qb, 0)),
            pl.BlockSpec((D_MODEL, DH), lambda qb, h: (0, h)),
            pl.BlockSpec((1, SKV, DH), lambda qb, h: (h, 0, 0)),
            pl.BlockSpec((1, SKV, DH), lambda qb, h: (h, 0, 0)),
            pl.BlockSpec((DH, D_MODEL), lambda qb, h: (h, 0)),
        ],
        out_specs=pl.BlockSpec((QBLK, D_MODEL), lambda qb, h: (qb, 0)),
        out_shape=jax.ShapeDtypeStruct((SQ, D_MODEL), jnp.float32),
    )(x2d, Wq_loc, K, V, Wo_loc)

    if __import__("os").environ.get("SKIP_AR"):
        return partial.reshape(1, SQ, D_MODEL)
    out = pl.pallas_call(
        _allreduce_body,
        out_shape=jax.ShapeDtypeStruct((SQ, D_MODEL), jnp.float32),
        in_specs=[pl.BlockSpec(memory_space=pltpu.VMEM)],
        out_specs=pl.BlockSpec(memory_space=pltpu.VMEM),
        scratch_shapes=[
            pltpu.VMEM((2, CHUNK, D_MODEL), jnp.float32),
            pltpu.SemaphoreType.DMA((2,)),
            pltpu.SemaphoreType.DMA((2,)),
        ],
    )(partial)
    return out.reshape(1, SQ, D_MODEL)


# device time: 248976 ns/iter; 2.1249x vs baseline; 1.5916x over previous
import jax
import jax.numpy as jnp
from jax import lax
from jax.experimental import pallas as pl
from jax.experimental.pallas import tpu as pltpu

N_DEV = 4
SQ = 2048
SKV = 2048
D_MODEL = 1024
H_PER = 8
DH = 128
CHUNK = SQ // N_DEV
SCALE = 0.08838834764831843


def _fused_body(x_ref, wq_ref, k_ref, v_ref, wo_ref, out_ref,
                bias_ref, comm_ref, send_sems, recv_sems, credit_sem):
    my = lax.axis_index("i")
    right = (my + 1) % N_DEV
    left = (my - 1) % N_DEV

    def compute_chunk(c):
        rows = pl.ds(c * CHUNK, CHUNK)
        row = c * CHUNK + lax.broadcasted_iota(jnp.int32, (CHUNK, SKV), 0)
        col = lax.broadcasted_iota(jnp.int32, (CHUNK, SKV), 1)
        qb = row // 64
        kb = col // 64
        mask = (qb == kb) | (kb == 0) | (((qb + kb) % 3) == 0)
        bias_ref[...] = jnp.where(mask, 0.0, -1e9)

        xb = x_ref[rows, :]
        acc = jnp.zeros((CHUNK, D_MODEL), jnp.float32)
        for h in range(H_PER):
            q = jnp.dot(xb, wq_ref[:, h * DH:(h + 1) * DH],
                        preferred_element_type=jnp.float32)
            s = lax.dot_general(
                q, k_ref[h], (((1,), (1,)), ((), ())),
                preferred_element_type=jnp.float32,
            ) * SCALE + bias_ref[...]
            m = jnp.max(s, axis=1, keepdims=True)
            w = jnp.exp(s - m)
            w = w / jnp.sum(w, axis=1, keepdims=True)
            ctx = jnp.dot(w, v_ref[h], preferred_element_type=jnp.float32)
            acc = acc + jnp.dot(ctx, wo_ref[h * DH:(h + 1) * DH, :],
                                preferred_element_type=jnp.float32)
        out_ref[rows, :] = acc

    sends = []

    c0 = (my + 3) % N_DEV
    compute_chunk(c0)
    send0 = pltpu.make_async_remote_copy(
        src_ref=out_ref.at[pl.ds(c0 * CHUNK, CHUNK), :],
        dst_ref=comm_ref.at[0],
        send_sem=send_sems.at[0],
        recv_sem=recv_sems.at[0],
        device_id=(right,),
        device_id_type=pl.DeviceIdType.MESH,
    )
    send0.start()
    sends.append(send0)

    for s in range(1, N_DEV):
        c = (my + 3 - s) % N_DEV
        compute_chunk(c)
        rows = pl.ds(c * CHUNK, CHUNK)
        slot = (s - 1) % 2
        recv = pltpu.make_async_remote_copy(
            src_ref=comm_ref.at[slot],
            dst_ref=comm_ref.at[slot],
            send_sem=send_sems.at[slot],
            recv_sem=recv_sems.at[slot],
            device_id=(left,),
            device_id_type=pl.DeviceIdType.MESH,
        )
        recv.wait_recv()
        out_ref[rows, :] += comm_ref[slot]
        if s == 1:
            pl.semaphore_signal(
                credit_sem, inc=1,
                device_id=(left,), device_id_type=pl.DeviceIdType.MESH,
            )
        if s < N_DEV - 1:
            if s == 2:
                pl.semaphore_wait(credit_sem, 1)
                send0.wait_send()
            snd = pltpu.make_async_remote_copy(
                src_ref=out_ref.at[rows, :],
                dst_ref=comm_ref.at[s % 2],
                send_sem=send_sems.at[s % 2],
                recv_sem=recv_sems.at[s % 2],
                device_id=(right,),
                device_id_type=pl.DeviceIdType.MESH,
            )
            snd.start()
            sends.append(snd)

    sends[1].wait_send()
    sends[2].wait_send()


def _ag_body(p_ref, out_ref, send_sems, recv_sems):
    my = lax.axis_index("i")
    right = (my + 1) % N_DEV

    out_ref[...] = p_ref[...]
    for t in range(N_DEV - 1):
        c = (my - t) % N_DEV
        slot = t % 2
        rdma = pltpu.make_async_remote_copy(
            src_ref=out_ref.at[pl.ds(c * CHUNK, CHUNK), :],
            dst_ref=out_ref.at[pl.ds(c * CHUNK, CHUNK), :],
            send_sem=send_sems.at[slot],
            recv_sem=recv_sems.at[slot],
            device_id=(right,),
            device_id_type=pl.DeviceIdType.MESH,
        )
        rdma.start()
        rdma.wait()


def kernel(x, Wq, K_ext, V_ext, Wo):
    my = lax.axis_index("i")
    x2d = x.reshape(SQ, D_MODEL)
    Wq_loc = lax.dynamic_slice(Wq, (0, my * (H_PER * DH)), (D_MODEL, H_PER * DH))
    Wo_loc = lax.dynamic_slice(Wo, (my * (H_PER * DH), 0), (H_PER * DH, D_MODEL))
    K = K_ext.reshape(SKV, H_PER, DH).transpose(1, 0, 2)
    V = V_ext.reshape(SKV, H_PER, DH).transpose(1, 0, 2)

    reduced = pl.pallas_call(
        _fused_body,
        out_shape=jax.ShapeDtypeStruct((SQ, D_MODEL), jnp.float32),
        in_specs=[pl.BlockSpec(memory_space=pltpu.VMEM)] * 5,
        out_specs=pl.BlockSpec(memory_space=pltpu.VMEM),
        scratch_shapes=[
            pltpu.VMEM((CHUNK, SKV), jnp.float32),
            pltpu.VMEM((2, CHUNK, D_MODEL), jnp.float32),
            pltpu.SemaphoreType.DMA((2,)),
            pltpu.SemaphoreType.DMA((2,)),
            pltpu.SemaphoreType.REGULAR,
        ],
        compiler_params=pltpu.CompilerParams(
            vmem_limit_bytes=100 * 1024 * 1024,
        ),
    )(x2d, Wq_loc, K, V, Wo_loc)

    out = pl.pallas_call(
        _ag_body,
        out_shape=jax.ShapeDtypeStruct((SQ, D_MODEL), jnp.float32),
        in_specs=[pl.BlockSpec(memory_space=pltpu.VMEM)],
        out_specs=pl.BlockSpec(memory_space=pltpu.VMEM),
        scratch_shapes=[
            pltpu.SemaphoreType.DMA((2,)),
            pltpu.SemaphoreType.DMA((2,)),
        ],
    )(reduced)
    return out.reshape(1, SQ, D_MODEL)


# device time: 225631 ns/iter; 2.3447x vs baseline; 1.1035x over previous
import jax
import jax.numpy as jnp
from jax import lax
from jax.experimental import pallas as pl
from jax.experimental.pallas import tpu as pltpu

N_DEV = 4
SQ = 2048
SKV = 2048
D_MODEL = 1024
H_PER = 8
DH = 128
CHUNK = SQ // N_DEV
HALF = CHUNK // 2
SCALE = 0.08838834764831843


def _fused_body(x_ref, wq_ref, k_ref, v_ref, wo_ref, out_ref,
                bias_ref, comm_ref, send_sems, recv_sems, credit_sem):
    my = lax.axis_index("i")
    right = (my + 1) % N_DEV
    left = (my - 1) % N_DEV

    def compute_chunk(c):
        rows = pl.ds(c * CHUNK, CHUNK)
        row = c * CHUNK + lax.broadcasted_iota(jnp.int32, (CHUNK, SKV), 0)
        col = lax.broadcasted_iota(jnp.int32, (CHUNK, SKV), 1)
        qb = row // 64
        kb = col // 64
        mask = (qb == kb) | (kb == 0) | (((qb + kb) % 3) == 0)
        bias_ref[...] = jnp.where(mask, 0.0, -1e9)

        xb = x_ref[rows, :]
        out_ref[rows, :] = jnp.zeros((CHUNK, D_MODEL), jnp.float32)

        def h_body(h, _):
            q = jnp.dot(xb, wq_ref[h], preferred_element_type=jnp.float32)
            s = lax.dot_general(
                q, k_ref[h], (((1,), (1,)), ((), ())),
                preferred_element_type=jnp.float32,
            ) * SCALE + bias_ref[...]
            m = jnp.max(s, axis=1, keepdims=True)
            w = jnp.exp(s - m)
            w = w / jnp.sum(w, axis=1, keepdims=True)
            ctx = jnp.dot(w, v_ref[h], preferred_element_type=jnp.float32)
            out_ref[rows, :] += jnp.dot(ctx, wo_ref[h],
                                        preferred_element_type=jnp.float32)
            return _

        lax.fori_loop(0, H_PER, h_body, None)

    sends = []

    c0 = (my + 3) % N_DEV
    compute_chunk(c0)
    send0 = pltpu.make_async_remote_copy(
        src_ref=out_ref.at[pl.ds(c0 * CHUNK, CHUNK), :],
        dst_ref=comm_ref.at[0],
        send_sem=send_sems.at[0],
        recv_sem=recv_sems.at[0],
        device_id=(right,),
        device_id_type=pl.DeviceIdType.MESH,
    )
    send0.start()
    sends.append(send0)

    for s in range(1, N_DEV):
        c = (my + 3 - s) % N_DEV
        compute_chunk(c)
        rows = pl.ds(c * CHUNK, CHUNK)
        slot = (s - 1) % 2
        recv = pltpu.make_async_remote_copy(
            src_ref=comm_ref.at[slot],
            dst_ref=comm_ref.at[slot],
            send_sem=send_sems.at[slot],
            recv_sem=recv_sems.at[slot],
            device_id=(left,),
            device_id_type=pl.DeviceIdType.MESH,
        )
        recv.wait_recv()
        out_ref[rows, :] += comm_ref[slot]
        if s == 1:
            pl.semaphore_signal(
                credit_sem, inc=1,
                device_id=(left,), device_id_type=pl.DeviceIdType.MESH,
            )
        if s < N_DEV - 1:
            if s == 2:
                pl.semaphore_wait(credit_sem, 1)
                send0.wait_send()
            snd = pltpu.make_async_remote_copy(
                src_ref=out_ref.at[rows, :],
                dst_ref=comm_ref.at[s % 2],
                send_sem=send_sems.at[s % 2],
                recv_sem=recv_sems.at[s % 2],
                device_id=(right,),
                device_id_type=pl.DeviceIdType.MESH,
            )
            snd.start()
            sends.append(snd)

    sends[1].wait_send()
    sends[2].wait_send()


def _ag_body(p_ref, out_ref, sr_send, sr_recv, sl_send, sl_recv):
    my = lax.axis_index("i")
    right = (my + 1) % N_DEV
    left = (my + 3) % N_DEV

    out_ref[...] = p_ref[...]
    for t in range(N_DEV - 1):
        cr = (my - t) % N_DEV
        cl = (my + t) % N_DEV
        slot = t % 2
        ra = pltpu.make_async_remote_copy(
            src_ref=out_ref.at[pl.ds(cr * CHUNK, HALF), :],
            dst_ref=out_ref.at[pl.ds(cr * CHUNK, HALF), :],
            send_sem=sr_send.at[slot],
            recv_sem=sr_recv.at[slot],
            device_id=(right,),
            device_id_type=pl.DeviceIdType.MESH,
        )
        rb = pltpu.make_async_remote_copy(
            src_ref=out_ref.at[pl.ds(cl * CHUNK + HALF, HALF), :],
            dst_ref=out_ref.at[pl.ds(cl * CHUNK + HALF, HALF), :],
            send_sem=sl_send.at[slot],
            recv_sem=sl_recv.at[slot],
            device_id=(left,),
            device_id_type=pl.DeviceIdType.MESH,
        )
        ra.start()
        rb.start()
        ra.wait()
        rb.wait()


def kernel(x, Wq, K_ext, V_ext, Wo):
    my = lax.axis_index("i")
    x2d = x.reshape(SQ, D_MODEL)
    Wq_loc = lax.dynamic_slice(Wq, (0, my * (H_PER * DH)), (D_MODEL, H_PER * DH))
    Wo_loc = lax.dynamic_slice(Wo, (my * (H_PER * DH), 0), (H_PER * DH, D_MODEL))
    Wq_h = Wq_loc.reshape(D_MODEL, H_PER, DH).transpose(1, 0, 2)
    Wo_h = Wo_loc.reshape(H_PER, DH, D_MODEL)
    K = K_ext.reshape(SKV, H_PER, DH).transpose(1, 0, 2)
    V = V_ext.reshape(SKV, H_PER, DH).transpose(1, 0, 2)

    reduced = pl.pallas_call(
        _fused_body,
        out_shape=jax.ShapeDtypeStruct((SQ, D_MODEL), jnp.float32),
        in_specs=[pl.BlockSpec(memory_space=pltpu.VMEM)] * 5,
        out_specs=pl.BlockSpec(memory_space=pltpu.VMEM),
        scratch_shapes=[
            pltpu.VMEM((CHUNK, SKV), jnp.float32),
            pltpu.VMEM((2, CHUNK, D_MODEL), jnp.float32),
            pltpu.SemaphoreType.DMA((2,)),
            pltpu.SemaphoreType.DMA((2,)),
            pltpu.SemaphoreType.REGULAR,
        ],
        compiler_params=pltpu.CompilerParams(
            vmem_limit_bytes=100 * 1024 * 1024,
        ),
    )(x2d, Wq_h, K, V, Wo_h)

    out = pl.pallas_call(
        _ag_body,
        out_shape=jax.ShapeDtypeStruct((SQ, D_MODEL), jnp.float32),
        in_specs=[pl.BlockSpec(memory_space=pltpu.VMEM)],
        out_specs=pl.BlockSpec(memory_space=pltpu.VMEM),
        scratch_shapes=[
            pltpu.SemaphoreType.DMA((2,)),
            pltpu.SemaphoreType.DMA((2,)),
            pltpu.SemaphoreType.DMA((2,)),
            pltpu.SemaphoreType.DMA((2,)),
        ],
    )(reduced)
    return out.reshape(1, SQ, D_MODEL)


# device time: 189566 ns/iter; 2.7908x vs baseline; 1.1903x over previous
import jax
import jax.numpy as jnp
from jax import lax
from jax.experimental import pallas as pl
from jax.experimental.pallas import tpu as pltpu

N_DEV = 4
SQ = 2048
SKV = 2048
D_MODEL = 1024
H_PER = 8
DH = 128
CHUNK = SQ // N_DEV
HALF = CHUNK // 2
SCALE = 0.08838834764831843


def _fused_body(x_ref, wq_ref, k_ref, v_ref, wo_ref, out_ref,
                bias_ref, comm_ref, send_sems, recv_sems, credit_sem):
    my = lax.axis_index("i")
    right = (my + 1) % N_DEV
    left = (my - 1) % N_DEV

    def compute_chunk(c):
        rows = pl.ds(c * CHUNK, CHUNK)
        row = c * CHUNK + lax.broadcasted_iota(jnp.int32, (CHUNK, SKV), 0)
        col = lax.broadcasted_iota(jnp.int32, (CHUNK, SKV), 1)
        qb = row // 64
        kb = col // 64
        mask = (qb == kb) | (kb == 0) | (((qb + kb) % 3) == 0)
        bias_ref[...] = jnp.where(mask, 0.0, -1e9)

        xb = x_ref[rows, :]
        out_ref[rows, :] = jnp.zeros((CHUNK, D_MODEL), jnp.float32)

        def h_body(h, _):
            q = jnp.dot(xb, wq_ref[h], preferred_element_type=jnp.float32)
            s = lax.dot_general(
                q, k_ref[h], (((1,), (1,)), ((), ())),
                preferred_element_type=jnp.float32,
            ) + bias_ref[...]
            w = jnp.exp(s)
            denom = jnp.sum(w, axis=1, keepdims=True)
            ctx = jnp.dot(w, v_ref[h], preferred_element_type=jnp.float32)
            ctx = ctx / denom
            out_ref[rows, :] += jnp.dot(ctx, wo_ref[h],
                                        preferred_element_type=jnp.float32)
            return _

        lax.fori_loop(0, H_PER, h_body, None)

    sends = []

    c0 = (my + 3) % N_DEV
    compute_chunk(c0)
    send0 = pltpu.make_async_remote_copy(
        src_ref=out_ref.at[pl.ds(c0 * CHUNK, CHUNK), :],
        dst_ref=comm_ref.at[0],
        send_sem=send_sems.at[0],
        recv_sem=recv_sems.at[0],
        device_id=(right,),
        device_id_type=pl.DeviceIdType.MESH,
    )
    send0.start()
    sends.append(send0)

    for s in range(1, N_DEV):
        c = (my + 3 - s) % N_DEV
        compute_chunk(c)
        rows = pl.ds(c * CHUNK, CHUNK)
        slot = (s - 1) % 2
        recv = pltpu.make_async_remote_copy(
            src_ref=comm_ref.at[slot],
            dst_ref=comm_ref.at[slot],
            send_sem=send_sems.at[slot],
            recv_sem=recv_sems.at[slot],
            device_id=(left,),
            device_id_type=pl.DeviceIdType.MESH,
        )
        recv.wait_recv()
        out_ref[rows, :] += comm_ref[slot]
        if s == 1:
            pl.semaphore_signal(
                credit_sem, inc=1,
                device_id=(left,), device_id_type=pl.DeviceIdType.MESH,
            )
        if s < N_DEV - 1:
            if s == 2:
                pl.semaphore_wait(credit_sem, 1)
                send0.wait_send()
            snd = pltpu.make_async_remote_copy(
                src_ref=out_ref.at[rows, :],
                dst_ref=comm_ref.at[s % 2],
                send_sem=send_sems.at[s % 2],
                recv_sem=recv_sems.at[s % 2],
                device_id=(right,),
                device_id_type=pl.DeviceIdType.MESH,
            )
            snd.start()
            sends.append(snd)

    sends[1].wait_send()
    sends[2].wait_send()


def _ag_body(p_ref, out_ref, sr_send, sr_recv, sl_send, sl_recv):
    my = lax.axis_index("i")
    right = (my + 1) % N_DEV
    left = (my + 3) % N_DEV

    out_ref[...] = p_ref[...]
    for t in range(N_DEV - 1):
        cr = (my - t) % N_DEV
        cl = (my + t) % N_DEV
        slot = t % 2
        ra = pltpu.make_async_remote_copy(
            src_ref=out_ref.at[pl.ds(cr * CHUNK, HALF), :],
            dst_ref=out_ref.at[pl.ds(cr * CHUNK, HALF), :],
            send_sem=sr_send.at[slot],
            recv_sem=sr_recv.at[slot],
            device_id=(right,),
            device_id_type=pl.DeviceIdType.MESH,
        )
        rb = pltpu.make_async_remote_copy(
            src_ref=out_ref.at[pl.ds(cl * CHUNK + HALF, HALF), :],
            dst_ref=out_ref.at[pl.ds(cl * CHUNK + HALF, HALF), :],
            send_sem=sl_send.at[slot],
            recv_sem=sl_recv.at[slot],
            device_id=(left,),
            device_id_type=pl.DeviceIdType.MESH,
        )
        ra.start()
        rb.start()
        ra.wait()
        rb.wait()


def kernel(x, Wq, K_ext, V_ext, Wo):
    my = lax.axis_index("i")
    x2d = x.reshape(SQ, D_MODEL)
    Wq_loc = lax.dynamic_slice(Wq, (0, my * (H_PER * DH)), (D_MODEL, H_PER * DH))
    Wo_loc = lax.dynamic_slice(Wo, (my * (H_PER * DH), 0), (H_PER * DH, D_MODEL))
    Wq_h = Wq_loc.reshape(D_MODEL, H_PER, DH).transpose(1, 0, 2) * SCALE
    Wo_h = Wo_loc.reshape(H_PER, DH, D_MODEL)
    K = K_ext.reshape(SKV, H_PER, DH).transpose(1, 0, 2)
    V = V_ext.reshape(SKV, H_PER, DH).transpose(1, 0, 2)

    reduced = pl.pallas_call(
        _fused_body,
        out_shape=jax.ShapeDtypeStruct((SQ, D_MODEL), jnp.float32),
        in_specs=[pl.BlockSpec(memory_space=pltpu.VMEM)] * 5,
        out_specs=pl.BlockSpec(memory_space=pltpu.VMEM),
        scratch_shapes=[
            pltpu.VMEM((CHUNK, SKV), jnp.float32),
            pltpu.VMEM((2, CHUNK, D_MODEL), jnp.float32),
            pltpu.SemaphoreType.DMA((2,)),
            pltpu.SemaphoreType.DMA((2,)),
            pltpu.SemaphoreType.REGULAR,
        ],
        compiler_params=pltpu.CompilerParams(
            vmem_limit_bytes=100 * 1024 * 1024,
        ),
    )(x2d, Wq_h, K, V, Wo_h)

    out = pl.pallas_call(
        _ag_body,
        out_shape=jax.ShapeDtypeStruct((SQ, D_MODEL), jnp.float32),
        in_specs=[pl.BlockSpec(memory_space=pltpu.VMEM)],
        out_specs=pl.BlockSpec(memory_space=pltpu.VMEM),
        scratch_shapes=[
            pltpu.SemaphoreType.DMA((2,)),
            pltpu.SemaphoreType.DMA((2,)),
            pltpu.SemaphoreType.DMA((2,)),
            pltpu.SemaphoreType.DMA((2,)),
        ],
    )(reduced)
    return out.reshape(1, SQ, D_MODEL)
